# Initial kernel scaffold; baseline (speedup 1.0000x reference)
import jax
import jax.numpy as jnp
from jax.experimental import pallas as pl


def kernel(values, segment_ids):
    raise NotImplementedError("write your pallas kernel here")


# speedup vs baseline: 282.1358x; 282.1358x over previous
import functools

import jax
import jax.numpy as jnp
from jax import lax
from jax.experimental import pallas as pl
from jax.experimental.pallas import tpu as pltpu
from jax.experimental.pallas import tpu_sc as plsc

NNZ = 13421772
NUM_SEG = 8 * 4096

NW = 32
BUF = 8192
NBUF_MAIN = 51
CHUNK = NBUF_MAIN * BUF
NNZ_MAIN = NW * CHUNK
TAIL_PAD = NW * BUF
NVEC = BUF // 16
IDS_OFF = 16
IDS_BUF = IDS_OFF + BUF + 16


def _sc_segsum(vals_hbm, ids_hbm, tvals_hbm, tids_hbm, part_hbm,
               vb0, vb1, ib0, ib1, acc, sem0, sem1):
    wid = lax.axis_index("c") * 16 + lax.axis_index("s")
    base = wid * CHUNK

    def start_dma(g, vb, ib, sem):
        @pl.when(g < NBUF_MAIN)
        def _():
            off = base + g * BUF
            pltpu.async_copy(vals_hbm.at[pl.ds(off, BUF)], vb, sem)
            pltpu.async_copy(ids_hbm.at[pl.ds(off, BUF)],
                             ib.at[pl.ds(IDS_OFF, BUF)], sem)

        @pl.when(g == NBUF_MAIN)
        def _():
            off = wid * BUF
            pltpu.async_copy(tvals_hbm.at[pl.ds(off, BUF)], vb, sem)
            pltpu.async_copy(tids_hbm.at[pl.ds(off, BUF)],
                             ib.at[pl.ds(IDS_OFF, BUF)], sem)

    def drain(vb, ib, sem):
        pltpu.make_async_copy(vals_hbm.at[pl.ds(0, BUF)], vb, sem).wait()
        pltpu.make_async_copy(ids_hbm.at[pl.ds(0, BUF)],
                              ib.at[pl.ds(IDS_OFF, BUF)], sem).wait()

    def compute(vb, ib):
        lane = lax.iota(jnp.int32, 16)
        first_l = lane == 0
        last_l = lane == 15
        def body(j4, carry):
            for u in range(4):
                j = j4 * 4 + u
                off = IDS_OFF + j * 16
                v = vb[pl.ds(j * 16, 16)]
                sid = ib[pl.ds(off, 16)]
                sidp = ib[pl.ds(off - 1, 16)]
                sidn = ib[pl.ds(off + 1, 16)]
                c = jnp.cumsum(v)
                start_m = (sid != sidp) | first_l
                end_m = (sid != sidn) | last_l
                plsc.addupdate_scatter(acc, [sid], c, mask=end_m)
                plsc.addupdate_scatter(acc, [sid], v - c, mask=start_m)
            return carry

        lax.fori_loop(0, NVEC // 4, body, 0)

    zero = jnp.zeros((16,), jnp.float32)
    def zbody(i, carry):
        for u in range(4):
            acc[pl.ds((i * 4 + u) * 16, 16)] = zero
        return carry
    lax.fori_loop(0, NUM_SEG // 64, zbody, 0)

    sent = jnp.full((16,), -1, jnp.int32)
    ib0[pl.ds(0, 16)] = sent
    ib0[pl.ds(IDS_OFF + BUF, 16)] = sent
    ib1[pl.ds(0, 16)] = sent
    ib1[pl.ds(IDS_OFF + BUF, 16)] = sent

    start_dma(0, vb0, ib0, sem0)

    def pair(p, carry):
        g1 = 2 * p + 1
        drain(vb0, ib0, sem0)
        start_dma(g1, vb1, ib1, sem1)
        compute(vb0, ib0)
        drain(vb1, ib1, sem1)

        @pl.when(g1 + 1 < NBUF_MAIN + 1)
        def _():
            start_dma(g1 + 1, vb0, ib0, sem0)

        compute(vb1, ib1)
        return carry

    lax.fori_loop(0, (NBUF_MAIN + 1) // 2, pair, 0)

    pltpu.sync_copy(acc, part_hbm.at[wid])


def _tc_reduce(x_ref, o_ref):
    o_ref[...] = jnp.sum(x_ref[...], axis=0)


@jax.jit
def kernel(values, segment_ids):
    tail_v = jnp.pad(values[NNZ_MAIN:], (0, TAIL_PAD - (NNZ - NNZ_MAIN)))
    tail_i = jnp.pad(segment_ids[NNZ_MAIN:], (0, TAIL_PAD - (NNZ - NNZ_MAIN)))

    mesh = plsc.VectorSubcoreMesh(core_axis_name="c", subcore_axis_name="s")
    sc = pl.kernel(
        _sc_segsum,
        mesh=mesh,
        compiler_params=pltpu.CompilerParams(needs_layout_passes=False),
        out_type=jax.ShapeDtypeStruct((NW, NUM_SEG), jnp.float32),
        scratch_types=[
            pltpu.VMEM((BUF,), jnp.float32),
            pltpu.VMEM((BUF,), jnp.float32),
            pltpu.VMEM((IDS_BUF,), jnp.int32),
            pltpu.VMEM((IDS_BUF,), jnp.int32),
            pltpu.VMEM((NUM_SEG,), jnp.float32),
            pltpu.SemaphoreType.DMA,
            pltpu.SemaphoreType.DMA,
        ],
    )
    part = sc(values, segment_ids, tail_v, tail_i)

    node = pl.pallas_call(
        _tc_reduce,
        out_shape=jax.ShapeDtypeStruct((NUM_SEG,), jnp.float32),
    )(part)
    return node.reshape(-1, 4096, 1)


# speedup vs baseline: 287.1086x; 1.0176x over previous
import functools

import jax
import jax.numpy as jnp
from jax import lax
from jax.experimental import pallas as pl
from jax.experimental.pallas import tpu as pltpu
from jax.experimental.pallas import tpu_sc as plsc

NNZ = 13421772
NUM_SEG = 8 * 4096

NW = 32
BUF = 8192
NBUF_MAIN = 51
CHUNK = NBUF_MAIN * BUF
NNZ_MAIN = NW * CHUNK
TAIL_PAD = NW * BUF
NVEC = BUF // 16
IDS_OFF = 16
IDS_BUF = IDS_OFF + BUF + 16


def _sc_segsum(vals_hbm, ids_hbm, tvals_hbm, tids_hbm, part_hbm,
               vb0, vb1, ib0, ib1, acc, sem0, sem1):
    wid = lax.axis_index("c") * 16 + lax.axis_index("s")
    base = wid * CHUNK

    def start_dma(g, vb, ib, sem):
        @pl.when(g < NBUF_MAIN)
        def _():
            off = base + g * BUF
            pltpu.async_copy(vals_hbm.at[pl.ds(off, BUF)], vb, sem)
            pltpu.async_copy(ids_hbm.at[pl.ds(off, BUF)],
                             ib.at[pl.ds(IDS_OFF, BUF)], sem)

        @pl.when(g == NBUF_MAIN)
        def _():
            off = wid * BUF
            pltpu.async_copy(tvals_hbm.at[pl.ds(off, BUF)], vb, sem)
            pltpu.async_copy(tids_hbm.at[pl.ds(off, BUF)],
                             ib.at[pl.ds(IDS_OFF, BUF)], sem)

    def drain(vb, ib, sem):
        pltpu.make_async_copy(vals_hbm.at[pl.ds(0, BUF)], vb, sem).wait()
        pltpu.make_async_copy(ids_hbm.at[pl.ds(0, BUF)],
                              ib.at[pl.ds(IDS_OFF, BUF)], sem).wait()

    def compute(vb, ib):
        lane = lax.iota(jnp.int32, 16)
        first_l = lane == 0
        last_l = lane == 15
        def body(j4, carry):
            for u in range(8):
                j = j4 * 8 + u
                off = IDS_OFF + j * 16
                v = vb[pl.ds(j * 16, 16)]
                sid = ib[pl.ds(off, 16)]
                sidp = ib[pl.ds(off - 1, 16)]
                sidn = ib[pl.ds(off + 1, 16)]
                c = jnp.cumsum(v)
                start_m = (sid != sidp) | first_l
                end_m = (sid != sidn) | last_l
                plsc.addupdate_scatter(acc, [sid], c, mask=end_m)
                plsc.addupdate_scatter(acc, [sid], v - c, mask=start_m)
            return carry

        lax.fori_loop(0, NVEC // 8, body, 0)

    zero = jnp.zeros((16,), jnp.float32)
    def zbody(i, carry):
        for u in range(4):
            acc[pl.ds((i * 4 + u) * 16, 16)] = zero
        return carry
    lax.fori_loop(0, NUM_SEG // 64, zbody, 0)

    sent = jnp.full((16,), -1, jnp.int32)
    ib0[pl.ds(0, 16)] = sent
    ib0[pl.ds(IDS_OFF + BUF, 16)] = sent
    ib1[pl.ds(0, 16)] = sent
    ib1[pl.ds(IDS_OFF + BUF, 16)] = sent

    start_dma(0, vb0, ib0, sem0)

    def pair(p, carry):
        g1 = 2 * p + 1
        drain(vb0, ib0, sem0)
        start_dma(g1, vb1, ib1, sem1)
        compute(vb0, ib0)
        drain(vb1, ib1, sem1)

        @pl.when(g1 + 1 < NBUF_MAIN + 1)
        def _():
            start_dma(g1 + 1, vb0, ib0, sem0)

        compute(vb1, ib1)
        return carry

    lax.fori_loop(0, (NBUF_MAIN + 1) // 2, pair, 0)

    pltpu.sync_copy(acc, part_hbm.at[wid])


def _tc_reduce(x_ref, o_ref):
    o_ref[...] = jnp.sum(x_ref[...], axis=0)


@jax.jit
def kernel(values, segment_ids):
    tail_v = jnp.pad(values[NNZ_MAIN:], (0, TAIL_PAD - (NNZ - NNZ_MAIN)))
    tail_i = jnp.pad(segment_ids[NNZ_MAIN:], (0, TAIL_PAD - (NNZ - NNZ_MAIN)))

    mesh = plsc.VectorSubcoreMesh(core_axis_name="c", subcore_axis_name="s")
    sc = pl.kernel(
        _sc_segsum,
        mesh=mesh,
        compiler_params=pltpu.CompilerParams(needs_layout_passes=False),
        out_type=jax.ShapeDtypeStruct((NW, NUM_SEG), jnp.float32),
        scratch_types=[
            pltpu.VMEM((BUF,), jnp.float32),
            pltpu.VMEM((BUF,), jnp.float32),
            pltpu.VMEM((IDS_BUF,), jnp.int32),
            pltpu.VMEM((IDS_BUF,), jnp.int32),
            pltpu.VMEM((NUM_SEG,), jnp.float32),
            pltpu.SemaphoreType.DMA,
            pltpu.SemaphoreType.DMA,
        ],
    )
    part = sc(values, segment_ids, tail_v, tail_i)

    node = pl.pallas_call(
        _tc_reduce,
        out_shape=jax.ShapeDtypeStruct((NUM_SEG,), jnp.float32),
    )(part)
    return node.reshape(-1, 4096, 1)


# speedup vs baseline: 708.6042x; 2.4681x over previous
import functools

import jax
import jax.numpy as jnp
from jax import lax
from jax.experimental import pallas as pl
from jax.experimental.pallas import tpu as pltpu
from jax.experimental.pallas import tpu_sc as plsc

NNZ = 13421772
NUM_SEG = 8 * 4096

NW = 32
BUF = 8192
NBUF_MAIN = 51
CHUNK = NBUF_MAIN * BUF
NNZ_MAIN = NW * CHUNK
TAIL_PAD = NW * BUF
NVEC = BUF // 16
IDS_OFF = 16
IDS_BUF = IDS_OFF + BUF + 16


def _sc_segsum(vals_hbm, ids_hbm, tvals_hbm, tids_hbm, part_hbm,
               vb0, vb1, ib0, ib1, acc, sem0, sem1):
    wid = lax.axis_index("c") * 16 + lax.axis_index("s")
    base = wid * CHUNK

    def start_dma(g, vb, ib, sem):
        @pl.when(g < NBUF_MAIN)
        def _():
            off = base + g * BUF
            pltpu.async_copy(vals_hbm.at[pl.ds(off, BUF)], vb, sem)
            pltpu.async_copy(ids_hbm.at[pl.ds(off, BUF)],
                             ib.at[pl.ds(IDS_OFF, BUF)], sem)

        @pl.when(g == NBUF_MAIN)
        def _():
            off = wid * BUF
            pltpu.async_copy(tvals_hbm.at[pl.ds(off, BUF)], vb, sem)
            pltpu.async_copy(tids_hbm.at[pl.ds(off, BUF)],
                             ib.at[pl.ds(IDS_OFF, BUF)], sem)

    def drain(vb, ib, sem):
        pltpu.make_async_copy(vals_hbm.at[pl.ds(0, BUF)], vb, sem).wait()
        pltpu.make_async_copy(ids_hbm.at[pl.ds(0, BUF)],
                              ib.at[pl.ds(IDS_OFF, BUF)], sem).wait()

    def compute(vb, ib):
        lane = lax.iota(jnp.int32, 16)
        first_l = lane == 0
        last_l = lane == 15
        @plsc.parallel_loop(0, NVEC, unroll=8)
        def _(j):
            off = IDS_OFF + j * 16
            v = vb[pl.ds(j * 16, 16)]
            sid = ib[pl.ds(off, 16)]
            sidp = ib[pl.ds(off - 1, 16)]
            sidn = ib[pl.ds(off + 1, 16)]
            c = jnp.cumsum(v)
            start_m = (sid != sidp) | first_l
            end_m = (sid != sidn) | last_l
            plsc.addupdate_scatter(acc, [sid], c, mask=end_m)
            plsc.addupdate_scatter(acc, [sid], v - c, mask=start_m)

    zero = jnp.zeros((16,), jnp.float32)
    def zbody(i, carry):
        for u in range(4):
            acc[pl.ds((i * 4 + u) * 16, 16)] = zero
        return carry
    lax.fori_loop(0, NUM_SEG // 64, zbody, 0)

    sent = jnp.full((16,), -1, jnp.int32)
    ib0[pl.ds(0, 16)] = sent
    ib0[pl.ds(IDS_OFF + BUF, 16)] = sent
    ib1[pl.ds(0, 16)] = sent
    ib1[pl.ds(IDS_OFF + BUF, 16)] = sent

    start_dma(0, vb0, ib0, sem0)

    def pair(p, carry):
        g1 = 2 * p + 1
        drain(vb0, ib0, sem0)
        start_dma(g1, vb1, ib1, sem1)
        compute(vb0, ib0)
        drain(vb1, ib1, sem1)

        @pl.when(g1 + 1 < NBUF_MAIN + 1)
        def _():
            start_dma(g1 + 1, vb0, ib0, sem0)

        compute(vb1, ib1)
        return carry

    lax.fori_loop(0, (NBUF_MAIN + 1) // 2, pair, 0)

    pltpu.sync_copy(acc, part_hbm.at[wid])


def _tc_reduce(x_ref, o_ref):
    o_ref[...] = jnp.sum(x_ref[...], axis=0)


@jax.jit
def kernel(values, segment_ids):
    tail_v = jnp.pad(values[NNZ_MAIN:], (0, TAIL_PAD - (NNZ - NNZ_MAIN)))
    tail_i = jnp.pad(segment_ids[NNZ_MAIN:], (0, TAIL_PAD - (NNZ - NNZ_MAIN)))

    mesh = plsc.VectorSubcoreMesh(core_axis_name="c", subcore_axis_name="s")
    sc = pl.kernel(
        _sc_segsum,
        mesh=mesh,
        compiler_params=pltpu.CompilerParams(needs_layout_passes=False),
        out_type=jax.ShapeDtypeStruct((NW, NUM_SEG), jnp.float32),
        scratch_types=[
            pltpu.VMEM((BUF,), jnp.float32),
            pltpu.VMEM((BUF,), jnp.float32),
            pltpu.VMEM((IDS_BUF,), jnp.int32),
            pltpu.VMEM((IDS_BUF,), jnp.int32),
            pltpu.VMEM((NUM_SEG,), jnp.float32),
            pltpu.SemaphoreType.DMA,
            pltpu.SemaphoreType.DMA,
        ],
    )
    part = sc(values, segment_ids, tail_v, tail_i)

    node = pl.pallas_call(
        _tc_reduce,
        out_shape=jax.ShapeDtypeStruct((NUM_SEG,), jnp.float32),
    )(part)
    return node.reshape(-1, 4096, 1)


# speedup vs baseline: 879.9052x; 1.2417x over previous
import functools

import jax
import jax.numpy as jnp
from jax import lax
from jax.experimental import pallas as pl
from jax.experimental.pallas import tpu as pltpu
from jax.experimental.pallas import tpu_sc as plsc

NNZ = 13421772
NUM_SEG = 8 * 4096

NW = 32
BUF = 8192
NBUF_MAIN = 51
CHUNK = NBUF_MAIN * BUF
NNZ_MAIN = NW * CHUNK
TAIL_PAD = NW * BUF
NVEC = BUF // 16
IDS_BUF = BUF + 8


def _sc_segsum(vals_hbm, ids_hbm, tvals_hbm, tids_hbm, part_hbm,
               vb0, vb1, ib0, ib1, acc, sem0, sem1):
    wid = lax.axis_index("c") * 16 + lax.axis_index("s")
    base = wid * CHUNK

    def start_dma(g, vb, ib, sem):
        @pl.when(g < NBUF_MAIN)
        def _():
            off = base + g * BUF
            pltpu.async_copy(vals_hbm.at[pl.ds(off, BUF)], vb, sem)
            pltpu.async_copy(ids_hbm.at[pl.ds(off, BUF)],
                             ib.at[pl.ds(0, BUF)], sem)

        @pl.when(g == NBUF_MAIN)
        def _():
            off = wid * BUF
            pltpu.async_copy(tvals_hbm.at[pl.ds(off, BUF)], vb, sem)
            pltpu.async_copy(tids_hbm.at[pl.ds(off, BUF)],
                             ib.at[pl.ds(0, BUF)], sem)

    def drain(vb, ib, sem):
        pltpu.make_async_copy(vals_hbm.at[pl.ds(0, BUF)], vb, sem).wait()
        pltpu.make_async_copy(ids_hbm.at[pl.ds(0, BUF)],
                              ib.at[pl.ds(0, BUF)], sem).wait()

    def compute(vb, ib):
        lane = lax.iota(jnp.int32, 16)
        last_l = lane == 15
        shift = jnp.maximum(lane - 1, 0)

        @plsc.parallel_loop(0, NVEC, unroll=8)
        def _(j):
            off = j * 16
            v = vb[pl.ds(off, 16)]
            sid = ib[pl.ds(off, 16)]
            sidn = ib[pl.ds(off + 1, 16)]
            sidp = jnp.take_along_axis(
                sid, shift, axis=0, mode="promise_in_bounds")
            c = jnp.cumsum(v)
            start_m = sid != sidp
            end_m = (sid != sidn) | last_l
            plsc.addupdate_scatter(acc, [sid], c, mask=end_m)
            plsc.addupdate_scatter(acc, [sid], v - c, mask=start_m)

    zero = jnp.zeros((16,), jnp.float32)

    @plsc.parallel_loop(0, NUM_SEG // 16, unroll=8)
    def _(i):
        acc[pl.ds(i * 16, 16)] = zero

    start_dma(0, vb0, ib0, sem0)

    def pair(p, carry):
        g1 = 2 * p + 1
        drain(vb0, ib0, sem0)
        start_dma(g1, vb1, ib1, sem1)
        compute(vb0, ib0)
        drain(vb1, ib1, sem1)

        @pl.when(g1 + 1 < NBUF_MAIN + 1)
        def _():
            start_dma(g1 + 1, vb0, ib0, sem0)

        compute(vb1, ib1)
        return carry

    lax.fori_loop(0, (NBUF_MAIN + 1) // 2, pair, 0)

    pltpu.sync_copy(acc, part_hbm.at[wid])


def _tc_reduce(x_ref, o_ref):
    o_ref[...] = jnp.sum(x_ref[...], axis=0)


@jax.jit
def kernel(values, segment_ids):
    tail_v = jnp.pad(values[NNZ_MAIN:], (0, TAIL_PAD - (NNZ - NNZ_MAIN)))
    tail_i = jnp.pad(segment_ids[NNZ_MAIN:], (0, TAIL_PAD - (NNZ - NNZ_MAIN)))

    mesh = plsc.VectorSubcoreMesh(core_axis_name="c", subcore_axis_name="s")
    sc = pl.kernel(
        _sc_segsum,
        mesh=mesh,
        compiler_params=pltpu.CompilerParams(needs_layout_passes=False),
        out_type=jax.ShapeDtypeStruct((NW, NUM_SEG), jnp.float32),
        scratch_types=[
            pltpu.VMEM((BUF,), jnp.float32),
            pltpu.VMEM((BUF,), jnp.float32),
            pltpu.VMEM((IDS_BUF,), jnp.int32),
            pltpu.VMEM((IDS_BUF,), jnp.int32),
            pltpu.VMEM((NUM_SEG,), jnp.float32),
            pltpu.SemaphoreType.DMA,
            pltpu.SemaphoreType.DMA,
        ],
    )
    part = sc(values, segment_ids, tail_v, tail_i)

    node = pl.pallas_call(
        _tc_reduce,
        out_shape=jax.ShapeDtypeStruct((NUM_SEG,), jnp.float32),
    )(part)
    return node.reshape(-1, 4096, 1)


# speedup vs baseline: 940.1743x; 1.0685x over previous
import functools

import jax
import jax.numpy as jnp
from jax import lax
from jax.experimental import pallas as pl
from jax.experimental.pallas import tpu as pltpu
from jax.experimental.pallas import tpu_sc as plsc

NNZ = 13421772
NUM_SEG = 8 * 4096

NW = 32
BUF = 8192
NBUF_MAIN = 51
CHUNK = NBUF_MAIN * BUF
NNZ_MAIN = NW * CHUNK
TAIL_PAD = NW * BUF
NVEC = BUF // 16
IDS_BUF = BUF + 8


def _sc_segsum(vals_hbm, ids_hbm, tvals_hbm, tids_hbm, part_hbm,
               vb0, vb1, ib0, ib1, acc, sem0, sem1):
    wid = lax.axis_index("c") * 16 + lax.axis_index("s")
    base = wid * CHUNK

    def start_dma(g, vb, ib, sem):
        @pl.when(g < NBUF_MAIN)
        def _():
            off = base + g * BUF
            pltpu.async_copy(vals_hbm.at[pl.ds(off, BUF)], vb, sem)
            pltpu.async_copy(ids_hbm.at[pl.ds(off, BUF)],
                             ib.at[pl.ds(0, BUF)], sem)

        @pl.when(g == NBUF_MAIN)
        def _():
            off = wid * BUF
            pltpu.async_copy(tvals_hbm.at[pl.ds(off, BUF)], vb, sem)
            pltpu.async_copy(tids_hbm.at[pl.ds(off, BUF)],
                             ib.at[pl.ds(0, BUF)], sem)

    def drain(vb, ib, sem):
        pltpu.make_async_copy(vals_hbm.at[pl.ds(0, BUF)], vb, sem).wait()
        pltpu.make_async_copy(ids_hbm.at[pl.ds(0, BUF)],
                              ib.at[pl.ds(0, BUF)], sem).wait()

    def compute(vb, ib):
        lane = lax.iota(jnp.int32, 16)
        last_l = lane == 15
        shift_p = jnp.maximum(lane - 1, 0)
        shift_n = jnp.minimum(lane + 1, 15)

        @plsc.parallel_loop(0, NVEC // 2, unroll=4)
        def _(jj):
            for u in range(2):
                off = (jj * 2 + u) * 16
                v = vb[pl.ds(off, 16)]
                sid = ib[pl.ds(off, 16)]
                if u == 0:
                    sidn = ib[pl.ds(off + 1, 16)]
                else:
                    sidn = jnp.take_along_axis(
                        sid, shift_n, axis=0, mode="promise_in_bounds")
                sidp = jnp.take_along_axis(
                    sid, shift_p, axis=0, mode="promise_in_bounds")
                c = jnp.cumsum(v)
                start_m = sid != sidp
                end_m = (sid != sidn) | last_l
                plsc.addupdate_scatter(acc, [sid], c, mask=end_m)
                plsc.addupdate_scatter(acc, [sid], v - c, mask=start_m)

    zero = jnp.zeros((16,), jnp.float32)

    @plsc.parallel_loop(0, NUM_SEG // 16, unroll=8)
    def _(i):
        acc[pl.ds(i * 16, 16)] = zero

    start_dma(0, vb0, ib0, sem0)

    def pair(p, carry):
        g1 = 2 * p + 1
        drain(vb0, ib0, sem0)
        start_dma(g1, vb1, ib1, sem1)
        compute(vb0, ib0)
        drain(vb1, ib1, sem1)

        @pl.when(g1 + 1 < NBUF_MAIN + 1)
        def _():
            start_dma(g1 + 1, vb0, ib0, sem0)

        compute(vb1, ib1)
        return carry

    lax.fori_loop(0, (NBUF_MAIN + 1) // 2, pair, 0)

    pltpu.sync_copy(acc, part_hbm.at[wid])


def _tc_reduce(x_ref, o_ref):
    o_ref[...] = jnp.sum(x_ref[...], axis=0)


@jax.jit
def kernel(values, segment_ids):
    tail_v = jnp.pad(values[NNZ_MAIN:], (0, TAIL_PAD - (NNZ - NNZ_MAIN)))
    tail_i = jnp.pad(segment_ids[NNZ_MAIN:], (0, TAIL_PAD - (NNZ - NNZ_MAIN)))

    mesh = plsc.VectorSubcoreMesh(core_axis_name="c", subcore_axis_name="s")
    sc = pl.kernel(
        _sc_segsum,
        mesh=mesh,
        compiler_params=pltpu.CompilerParams(needs_layout_passes=False),
        out_type=jax.ShapeDtypeStruct((NW, NUM_SEG), jnp.float32),
        scratch_types=[
            pltpu.VMEM((BUF,), jnp.float32),
            pltpu.VMEM((BUF,), jnp.float32),
            pltpu.VMEM((IDS_BUF,), jnp.int32),
            pltpu.VMEM((IDS_BUF,), jnp.int32),
            pltpu.VMEM((NUM_SEG,), jnp.float32),
            pltpu.SemaphoreType.DMA,
            pltpu.SemaphoreType.DMA,
        ],
    )
    part = sc(values, segment_ids, tail_v, tail_i)

    node = pl.pallas_call(
        _tc_reduce,
        out_shape=jax.ShapeDtypeStruct((NUM_SEG,), jnp.float32),
    )(part)
    return node.reshape(-1, 4096, 1)
